# Initial kernel scaffold; baseline (speedup 1.0000x reference)
import jax
import jax.numpy as jnp
from jax.experimental import pallas as pl


def kernel(x, edge_index, W1, b1, W2, b2):
    raise NotImplementedError("write your pallas kernel here")


# speedup vs baseline: 13.4797x; 13.4797x over previous
import functools

import jax
import jax.numpy as jnp
from jax import lax
from jax.experimental import pallas as pl
from jax.experimental.pallas import tpu as pltpu
from jax.experimental.pallas import tpu_sc as plsc

N = 10000
E = 320000
D_IN = 128
D_H = 128
D_OUT = 40
D2P = 48

NC = 2
NS = 16
NW = NC * NS
EPW = E // NW
K = 80
NPAD = 10240
NPT = NPAD // NS
ZR = 128

_MESH = plsc.VectorSubcoreMesh(
    core_axis_name="c", subcore_axis_name="s", num_cores=NC, num_subcores=NS
)
_SC_PARAMS = pltpu.CompilerParams(use_tc_tiling_on_sc=False)


def _fill(ref, nrows, ncols, value):
    val = jnp.full((16,), value, jnp.float32)

    def body(i, _):
        for j in range(ncols // 16):
            ref[i, pl.ds(j * 16, 16)] = val
        return 0

    lax.fori_loop(0, nrows, body, 0)


def _zero_acc(acc, zbuf, s, d):
    _fill(zbuf, ZR, d, 0.0)
    for r in range(NPT // ZR):
        pltpu.sync_copy(zbuf, acc.at[pl.ds(s * NPT + r * ZR, ZR)])


@functools.partial(
    pl.kernel,
    out_type=jax.ShapeDtypeStruct((NC * NPAD, 16), jnp.float32),
    mesh=_MESH,
    scratch_types=[
        pltpu.VMEM((K,), jnp.int32),
        pltpu.VMEM((K, 16), jnp.float32),
        pltpu.VMEM((ZR, 16), jnp.float32),
        pltpu.VMEM_SHARED((NPAD, 16), jnp.float32),
    ],
    compiler_params=_SC_PARAMS,
)
def _deg_kernel(dst_hbm, out_hbm, dstbuf, ones_v, zbuf, acc):
    c = lax.axis_index("c")
    s = lax.axis_index("s")
    wid = c * NS + s
    _fill(ones_v, K, 16, 1.0)
    _zero_acc(acc, zbuf, s, 16)
    plsc.subcore_barrier()

    def body(i, _):
        base = wid * EPW + i * K
        pltpu.sync_copy(dst_hbm.at[pl.ds(base, K)], dstbuf)
        pltpu.sync_copy(ones_v, acc.at[dstbuf], add=True)
        return 0

    lax.fori_loop(0, EPW // K, body, 0)
    plsc.subcore_barrier()
    pltpu.sync_copy(
        acc.at[pl.ds(s * NPT, NPT)], out_hbm.at[pl.ds(c * NPAD + s * NPT, NPT)]
    )


def _make_prop(d):
    @functools.partial(
        pl.kernel,
        out_type=jax.ShapeDtypeStruct((NC * NPAD, d), jnp.float32),
        mesh=_MESH,
        scratch_types=[
            pltpu.VMEM((K,), jnp.int32),
            pltpu.VMEM((K,), jnp.int32),
            pltpu.VMEM((K, d), jnp.float32),
            pltpu.VMEM((ZR, d), jnp.float32),
            pltpu.VMEM_SHARED((NPAD, d), jnp.float32),
            pltpu.SemaphoreType.DMA,
        ],
        compiler_params=_SC_PARAMS,
    )
    def prop(y_hbm, src_hbm, dst_hbm, out_hbm, srcbuf, dstbuf, rows, zbuf, acc, sem):
        c = lax.axis_index("c")
        s = lax.axis_index("s")
        wid = c * NS + s
        _zero_acc(acc, zbuf, s, d)
        plsc.subcore_barrier()

        def body(i, _):
            base = wid * EPW + i * K
            pltpu.sync_copy(src_hbm.at[pl.ds(base, K)], srcbuf)
            pltpu.sync_copy(dst_hbm.at[pl.ds(base, K)], dstbuf)
            pltpu.async_copy(y_hbm.at[srcbuf], rows, sem).wait()
            pltpu.sync_copy(rows, acc.at[dstbuf], add=True)
            return 0

        lax.fori_loop(0, EPW // K, body, 0)
        plsc.subcore_barrier()
        pltpu.sync_copy(
            acc.at[pl.ds(s * NPT, NPT)], out_hbm.at[pl.ds(c * NPAD + s * NPT, NPT)]
        )

    return prop


_prop128 = _make_prop(D_H)
_prop48 = _make_prop(D2P)

R = 1000


def _tc1_body(dega_ref, degb_ref, x_ref, w1_ref, y1_ref, dinv_ref):
    deg = dega_ref[:, 0:1] + degb_ref[:, 0:1]
    dinv = lax.rsqrt(deg)
    xw = jnp.dot(x_ref[...], w1_ref[...], preferred_element_type=jnp.float32)
    y1_ref[...] = xw * dinv
    dinv_ref[...] = jnp.broadcast_to(dinv, (R, 8))


def _tc1(dega, degb, x, w1):
    return pl.pallas_call(
        _tc1_body,
        grid=(N // R,),
        in_specs=[
            pl.BlockSpec((R, 16), lambda i: (i, 0)),
            pl.BlockSpec((R, 16), lambda i: (i, 0)),
            pl.BlockSpec((R, D_IN), lambda i: (i, 0)),
            pl.BlockSpec((D_IN, D_H), lambda i: (0, 0)),
        ],
        out_specs=[
            pl.BlockSpec((R, D_H), lambda i: (i, 0)),
            pl.BlockSpec((R, 8), lambda i: (i, 0)),
        ],
        out_shape=[
            jax.ShapeDtypeStruct((N, D_H), jnp.float32),
            jax.ShapeDtypeStruct((N, 8), jnp.float32),
        ],
    )(dega, degb, x, w1)


def _tc2_body(acca_ref, accb_ref, y1_ref, dinv_ref, b1_ref, w2_ref, y2_ref):
    dinv = dinv_ref[:, 0:1]
    z = acca_ref[...] + accb_ref[...] + y1_ref[...]
    h = jnp.maximum(z * dinv + b1_ref[...], 0.0)
    y2_ref[...] = jnp.dot(h, w2_ref[...], preferred_element_type=jnp.float32) * dinv


def _tc2(acca, accb, y1, dinv, b1, w2p):
    return pl.pallas_call(
        _tc2_body,
        grid=(N // R,),
        in_specs=[
            pl.BlockSpec((R, D_H), lambda i: (i, 0)),
            pl.BlockSpec((R, D_H), lambda i: (i, 0)),
            pl.BlockSpec((R, D_H), lambda i: (i, 0)),
            pl.BlockSpec((R, 8), lambda i: (i, 0)),
            pl.BlockSpec((1, D_H), lambda i: (0, 0)),
            pl.BlockSpec((D_H, D2P), lambda i: (0, 0)),
        ],
        out_specs=pl.BlockSpec((R, D2P), lambda i: (i, 0)),
        out_shape=jax.ShapeDtypeStruct((N, D2P), jnp.float32),
    )(acca, accb, y1, dinv, b1, w2p)


def _tc3_body(acca_ref, accb_ref, y2_ref, dinv_ref, b2_ref, out_ref):
    dinv = dinv_ref[:, 0:1]
    z = (acca_ref[...] + accb_ref[...] + y2_ref[...]) * dinv
    o = z[:, :D_OUT] + b2_ref[...]
    m = jnp.max(o, axis=1, keepdims=True)
    e = jnp.exp(o - m)
    out_ref[...] = e / jnp.sum(e, axis=1, keepdims=True)


def _tc3(acca, accb, y2, dinv, b2):
    return pl.pallas_call(
        _tc3_body,
        grid=(N // R,),
        in_specs=[
            pl.BlockSpec((R, D2P), lambda i: (i, 0)),
            pl.BlockSpec((R, D2P), lambda i: (i, 0)),
            pl.BlockSpec((R, D2P), lambda i: (i, 0)),
            pl.BlockSpec((R, 8), lambda i: (i, 0)),
            pl.BlockSpec((1, D_OUT), lambda i: (0, 0)),
        ],
        out_specs=pl.BlockSpec((R, D_OUT), lambda i: (i, 0)),
        out_shape=jax.ShapeDtypeStruct((N, D_OUT), jnp.float32),
    )(acca, accb, y2, dinv, b2)


def kernel(x, edge_index, W1, b1, W2, b2):
    src = edge_index[0]
    dst = edge_index[1]
    w2p = jnp.pad(W2, ((0, 0), (0, D2P - D_OUT)))
    b1r = b1.reshape(1, D_H)
    b2r = b2.reshape(1, D_OUT)

    degp = _deg_kernel(dst)
    y1, dinv = _tc1(degp[:N], degp[NPAD:NPAD + N], x, W1)
    acc1 = _prop128(y1, src, dst)
    y2 = _tc2(acc1[:N], acc1[NPAD:NPAD + N], y1, dinv, b1r, w2p)
    acc2 = _prop48(y2, src, dst)
    return _tc3(acc2[:N], acc2[NPAD:NPAD + N], y2, dinv, b2r)


# speedup vs baseline: 24.4327x; 1.8125x over previous
import functools

import jax
import jax.numpy as jnp
import numpy as np
from jax import lax
from jax.experimental import pallas as pl
from jax.experimental.pallas import tpu as pltpu
from jax.experimental.pallas import tpu_sc as plsc

N = 10000
E = 320000
D_IN = 128
D_H = 128
DH2 = 64
D_OUT = 40
D2P = 48

NC = 2
NS = 16
NW = NC * NS
K = 128
CPW = 80
EP = NW * CPW * K
NPAD = 10112
NPT = NPAD // NS
ZR = 158

_MESH = plsc.VectorSubcoreMesh(
    core_axis_name="c", subcore_axis_name="s", num_cores=NC, num_subcores=NS
)
_SC_PARAMS = pltpu.CompilerParams(use_tc_tiling_on_sc=False)

_PAD_DST = np.asarray(N + (np.arange(EP - E) % (NPAD - N)), np.int32)
_PAD_SRC = np.asarray((np.arange(EP - E) * 37) % N, np.int32)


def _fill(ref, nrows, ncols, value):
    val = jnp.full((16,), value, jnp.float32)

    def body(i, _):
        for j in range(ncols // 16):
            ref[i, pl.ds(j * 16, 16)] = val
        return 0

    lax.fori_loop(0, nrows, body, 0)


def _zero_acc(acc, zbuf, s):
    for r in range(NPT // ZR):
        pltpu.sync_copy(zbuf, acc.at[pl.ds(s * NPT + r * ZR, ZR)])


def _scatter_pass(y_hbm, idxs, idxd, rows0, rows1, acc, sg0, sg1, ss0, ss1):
    def body(t, _):
        g0 = pltpu.async_copy(y_hbm.at[idxs.at[2 * t]], rows0, sg0)
        g1 = pltpu.async_copy(y_hbm.at[idxs.at[2 * t + 1]], rows1, sg1)
        g0.wait()
        s0 = pltpu.async_copy(rows0, acc.at[idxd.at[2 * t]], ss0, add=True)
        g1.wait()
        s1 = pltpu.async_copy(rows1, acc.at[idxd.at[2 * t + 1]], ss1, add=True)
        s0.wait()
        s1.wait()
        return 0

    lax.fori_loop(0, CPW // 2, body, 0)


@functools.partial(
    pl.kernel,
    out_type=jax.ShapeDtypeStruct((NC * NPAD, 16), jnp.float32),
    mesh=_MESH,
    scratch_types=[
        pltpu.VMEM((CPW, K), jnp.int32),
        pltpu.VMEM((K, 16), jnp.float32),
        pltpu.VMEM((ZR, 16), jnp.float32),
        pltpu.VMEM_SHARED((NPAD, 16), jnp.float32),
        pltpu.SemaphoreType.DMA,
    ],
    compiler_params=_SC_PARAMS,
)
def _deg_kernel(dst_hbm, out_hbm, idxd, ones_v, zbuf, acc, sem):
    c = lax.axis_index("c")
    s = lax.axis_index("s")
    wid = c * NS + s
    _fill(ones_v, K, 16, 1.0)
    _fill(zbuf, ZR, 16, 0.0)
    _zero_acc(acc, zbuf, s)
    plsc.subcore_barrier()
    pltpu.sync_copy(dst_hbm.at[pl.ds(wid * CPW, CPW)], idxd)

    def body(t, _):
        ds_ = [
            pltpu.async_copy(ones_v, acc.at[idxd.at[4 * t + u]], sem, add=True)
            for u in range(4)
        ]
        for d_ in ds_:
            d_.wait()
        return 0

    lax.fori_loop(0, CPW // 4, body, 0)
    plsc.subcore_barrier()
    pltpu.sync_copy(
        acc.at[pl.ds(s * NPT, NPT)], out_hbm.at[pl.ds(c * NPAD + s * NPT, NPT)]
    )


@functools.partial(
    pl.kernel,
    out_type=[
        jax.ShapeDtypeStruct((NC * NPAD, DH2), jnp.float32),
        jax.ShapeDtypeStruct((NC * NPAD, DH2), jnp.float32),
    ],
    mesh=_MESH,
    scratch_types=[
        pltpu.VMEM((CPW, K), jnp.int32),
        pltpu.VMEM((CPW, K), jnp.int32),
        pltpu.VMEM((K, DH2), jnp.float32),
        pltpu.VMEM((K, DH2), jnp.float32),
        pltpu.VMEM((ZR, DH2), jnp.float32),
        pltpu.VMEM_SHARED((NPAD, DH2), jnp.float32),
        pltpu.SemaphoreType.DMA,
        pltpu.SemaphoreType.DMA,
        pltpu.SemaphoreType.DMA,
        pltpu.SemaphoreType.DMA,
    ],
    compiler_params=_SC_PARAMS,
)
def _prop1(ya_hbm, yb_hbm, src_hbm, dst_hbm, outa_hbm, outb_hbm, idxs, idxd,
           rows0, rows1, zbuf, acc, sg0, sg1, ss0, ss1):
    c = lax.axis_index("c")
    s = lax.axis_index("s")
    wid = c * NS + s
    _fill(zbuf, ZR, DH2, 0.0)
    pltpu.sync_copy(src_hbm.at[pl.ds(wid * CPW, CPW)], idxs)
    pltpu.sync_copy(dst_hbm.at[pl.ds(wid * CPW, CPW)], idxd)
    for y_hbm, out_hbm in ((ya_hbm, outa_hbm), (yb_hbm, outb_hbm)):
        _zero_acc(acc, zbuf, s)
        plsc.subcore_barrier()
        _scatter_pass(y_hbm, idxs, idxd, rows0, rows1, acc, sg0, sg1, ss0, ss1)
        plsc.subcore_barrier()
        pltpu.sync_copy(
            acc.at[pl.ds(s * NPT, NPT)],
            out_hbm.at[pl.ds(c * NPAD + s * NPT, NPT)],
        )
        plsc.subcore_barrier()


@functools.partial(
    pl.kernel,
    out_type=jax.ShapeDtypeStruct((NC * NPAD, D2P), jnp.float32),
    mesh=_MESH,
    scratch_types=[
        pltpu.VMEM((CPW, K), jnp.int32),
        pltpu.VMEM((CPW, K), jnp.int32),
        pltpu.VMEM((K, D2P), jnp.float32),
        pltpu.VMEM((K, D2P), jnp.float32),
        pltpu.VMEM((ZR, D2P), jnp.float32),
        pltpu.VMEM_SHARED((NPAD, D2P), jnp.float32),
        pltpu.SemaphoreType.DMA,
        pltpu.SemaphoreType.DMA,
        pltpu.SemaphoreType.DMA,
        pltpu.SemaphoreType.DMA,
    ],
    compiler_params=_SC_PARAMS,
)
def _prop2(y_hbm, src_hbm, dst_hbm, out_hbm, idxs, idxd, rows0, rows1, zbuf,
           acc, sg0, sg1, ss0, ss1):
    c = lax.axis_index("c")
    s = lax.axis_index("s")
    wid = c * NS + s
    _fill(zbuf, ZR, D2P, 0.0)
    _zero_acc(acc, zbuf, s)
    plsc.subcore_barrier()
    pltpu.sync_copy(src_hbm.at[pl.ds(wid * CPW, CPW)], idxs)
    pltpu.sync_copy(dst_hbm.at[pl.ds(wid * CPW, CPW)], idxd)
    _scatter_pass(y_hbm, idxs, idxd, rows0, rows1, acc, sg0, sg1, ss0, ss1)
    plsc.subcore_barrier()
    pltpu.sync_copy(
        acc.at[pl.ds(s * NPT, NPT)], out_hbm.at[pl.ds(c * NPAD + s * NPT, NPT)]
    )


R = 1000


def _tc1_body(dega_ref, degb_ref, x_ref, w1_ref, ya_ref, yb_ref, dinv_ref):
    deg = dega_ref[:, 0:1] + degb_ref[:, 0:1]
    dinv = lax.rsqrt(deg)
    xw = jnp.dot(x_ref[...], w1_ref[...], preferred_element_type=jnp.float32)
    y1 = xw * dinv
    ya_ref[...] = y1[:, :DH2]
    yb_ref[...] = y1[:, DH2:]
    dinv_ref[...] = jnp.broadcast_to(dinv, (R, 8))


def _tc1(dega, degb, x, w1):
    return pl.pallas_call(
        _tc1_body,
        grid=(N // R,),
        in_specs=[
            pl.BlockSpec((R, 16), lambda i: (i, 0)),
            pl.BlockSpec((R, 16), lambda i: (i, 0)),
            pl.BlockSpec((R, D_IN), lambda i: (i, 0)),
            pl.BlockSpec((D_IN, D_H), lambda i: (0, 0)),
        ],
        out_specs=[
            pl.BlockSpec((R, DH2), lambda i: (i, 0)),
            pl.BlockSpec((R, DH2), lambda i: (i, 0)),
            pl.BlockSpec((R, 8), lambda i: (i, 0)),
        ],
        out_shape=[
            jax.ShapeDtypeStruct((N, DH2), jnp.float32),
            jax.ShapeDtypeStruct((N, DH2), jnp.float32),
            jax.ShapeDtypeStruct((N, 8), jnp.float32),
        ],
    )(dega, degb, x, w1)


def _tc2_body(za_a, za_b, zb_a, zb_b, ya_ref, yb_ref, dinv_ref, b1_ref, w2_ref,
              y2_ref):
    dinv = dinv_ref[:, 0:1]
    zlo = za_a[...] + za_b[...] + ya_ref[...]
    zhi = zb_a[...] + zb_b[...] + yb_ref[...]
    z = jnp.concatenate([zlo, zhi], axis=1)
    h = jnp.maximum(z * dinv + b1_ref[...], 0.0)
    y2_ref[...] = jnp.dot(h, w2_ref[...], preferred_element_type=jnp.float32) * dinv


def _tc2(acca, accb, ya, yb, dinv, b1, w2p):
    half = pl.BlockSpec((R, DH2), lambda i: (i, 0))
    return pl.pallas_call(
        _tc2_body,
        grid=(N // R,),
        in_specs=[
            half, half, half, half, half, half,
            pl.BlockSpec((R, 8), lambda i: (i, 0)),
            pl.BlockSpec((1, D_H), lambda i: (0, 0)),
            pl.BlockSpec((D_H, D2P), lambda i: (0, 0)),
        ],
        out_specs=pl.BlockSpec((R, D2P), lambda i: (i, 0)),
        out_shape=jax.ShapeDtypeStruct((N, D2P), jnp.float32),
    )(acca[:N], acca[NPAD:NPAD + N], accb[:N], accb[NPAD:NPAD + N], ya, yb,
      dinv, b1, w2p)


def _tc3_body(acca_ref, accb_ref, y2_ref, dinv_ref, b2_ref, out_ref):
    dinv = dinv_ref[:, 0:1]
    z = (acca_ref[...] + accb_ref[...] + y2_ref[...]) * dinv
    o = z[:, :D_OUT] + b2_ref[...]
    m = jnp.max(o, axis=1, keepdims=True)
    e = jnp.exp(o - m)
    out_ref[...] = e / jnp.sum(e, axis=1, keepdims=True)


def _tc3(acc2, y2, dinv, b2):
    return pl.pallas_call(
        _tc3_body,
        grid=(N // R,),
        in_specs=[
            pl.BlockSpec((R, D2P), lambda i: (i, 0)),
            pl.BlockSpec((R, D2P), lambda i: (i, 0)),
            pl.BlockSpec((R, D2P), lambda i: (i, 0)),
            pl.BlockSpec((R, 8), lambda i: (i, 0)),
            pl.BlockSpec((1, D_OUT), lambda i: (0, 0)),
        ],
        out_specs=pl.BlockSpec((R, D_OUT), lambda i: (i, 0)),
        out_shape=jax.ShapeDtypeStruct((N, D_OUT), jnp.float32),
    )(acc2[:N], acc2[NPAD:NPAD + N], y2, dinv, b2)


def kernel(x, edge_index, W1, b1, W2, b2):
    src = jnp.concatenate([edge_index[0], jnp.asarray(_PAD_SRC)])
    dst = jnp.concatenate([edge_index[1], jnp.asarray(_PAD_DST)])
    src2 = src.reshape(NW * CPW, K)
    dst2 = dst.reshape(NW * CPW, K)
    w2p = jnp.pad(W2, ((0, 0), (0, D2P - D_OUT)))
    b1r = b1.reshape(1, D_H)
    b2r = b2.reshape(1, D_OUT)

    degp = _deg_kernel(dst2)
    ya, yb, dinv = _tc1(degp[:N], degp[NPAD:NPAD + N], x, W1)
    acc1a, acc1b = _prop1(ya, yb, src2, dst2)
    y2 = _tc2(acc1a, acc1b, ya, yb, dinv, b1r, w2p)
    acc2 = _prop2(y2, src2, dst2)
    return _tc3(acc2, y2, dinv, b2r)


# speedup vs baseline: 26.4813x; 1.0838x over previous
import functools

import jax
import jax.numpy as jnp
import numpy as np
from jax import lax
from jax.experimental import pallas as pl
from jax.experimental.pallas import tpu as pltpu
from jax.experimental.pallas import tpu_sc as plsc

N = 10000
E = 320000
D_IN = 128
D_H = 128
DH2 = 64
D_OUT = 40
D2P = 48

NC = 2
NS = 16
NW = NC * NS
K = 128
CPW = 80
EP = NW * CPW * K
NPAD = 10112
NPT = NPAD // NS
ZR = 158

_MESH = plsc.VectorSubcoreMesh(
    core_axis_name="c", subcore_axis_name="s", num_cores=NC, num_subcores=NS
)
_SC_PARAMS = pltpu.CompilerParams(use_tc_tiling_on_sc=False)

_PAD_DST = np.asarray(N + (np.arange(EP - E) % (NPAD - N)), np.int32)
_PAD_SRC = np.asarray((np.arange(EP - E) * 37) % N, np.int32)


def _fill(ref, nrows, ncols, value):
    val = jnp.full((16,), value, jnp.float32)

    def body(i, _):
        for j in range(ncols // 16):
            ref[i, pl.ds(j * 16, 16)] = val
        return 0

    lax.fori_loop(0, nrows, body, 0)


def _zero_acc(acc, zbuf, s):
    for r in range(NPT // ZR):
        pltpu.sync_copy(zbuf, acc.at[pl.ds(s * NPT + r * ZR, ZR)])


NB = 4


def _scatter_pass(y_hbm, idxs, idxd, bufs, acc, sgs, sss):
    def body(t, _):
        gs = [
            pltpu.async_copy(y_hbm.at[idxs.at[NB * t + u]], bufs[u], sgs[u])
            for u in range(NB)
        ]
        ss = []
        for u in range(NB):
            gs[u].wait()
            ss.append(
                pltpu.async_copy(bufs[u], acc.at[idxd.at[NB * t + u]], sss[u],
                                 add=True)
            )
        for s_ in ss:
            s_.wait()
        return 0

    lax.fori_loop(0, CPW // NB, body, 0)


@functools.partial(
    pl.kernel,
    out_type=jax.ShapeDtypeStruct((NC * NPAD, 16), jnp.float32),
    mesh=_MESH,
    scratch_types=[
        pltpu.VMEM((CPW, K), jnp.int32),
        pltpu.VMEM((K, 16), jnp.float32),
        pltpu.VMEM((ZR, 16), jnp.float32),
        pltpu.VMEM_SHARED((NPAD, 16), jnp.float32),
        pltpu.SemaphoreType.DMA,
    ],
    compiler_params=_SC_PARAMS,
)
def _deg_kernel(dst_hbm, out_hbm, idxd, ones_v, zbuf, acc, sem):
    c = lax.axis_index("c")
    s = lax.axis_index("s")
    wid = c * NS + s
    _fill(ones_v, K, 16, 1.0)
    _fill(zbuf, ZR, 16, 0.0)
    _zero_acc(acc, zbuf, s)
    plsc.subcore_barrier()
    pltpu.sync_copy(dst_hbm.at[pl.ds(wid * CPW, CPW)], idxd)

    def body(t, _):
        ds_ = [
            pltpu.async_copy(ones_v, acc.at[idxd.at[4 * t + u]], sem, add=True)
            for u in range(4)
        ]
        for d_ in ds_:
            d_.wait()
        return 0

    lax.fori_loop(0, CPW // 4, body, 0)
    plsc.subcore_barrier()
    pltpu.sync_copy(
        acc.at[pl.ds(s * NPT, NPT)], out_hbm.at[pl.ds(c * NPAD + s * NPT, NPT)]
    )


@functools.partial(
    pl.kernel,
    out_type=[
        jax.ShapeDtypeStruct((NC * NPAD, DH2), jnp.float32),
        jax.ShapeDtypeStruct((NC * NPAD, DH2), jnp.float32),
    ],
    mesh=_MESH,
    scratch_types=[
        pltpu.VMEM((CPW, K), jnp.int32),
        pltpu.VMEM((CPW, K), jnp.int32),
        pltpu.VMEM((K, DH2), jnp.float32),
        pltpu.VMEM((K, DH2), jnp.float32),
        pltpu.VMEM((K, DH2), jnp.float32),
        pltpu.VMEM((K, DH2), jnp.float32),
        pltpu.VMEM((ZR, DH2), jnp.float32),
        pltpu.VMEM_SHARED((NPAD, DH2), jnp.float32),
    ] + [pltpu.SemaphoreType.DMA] * 8,
    compiler_params=_SC_PARAMS,
)
def _prop1(ya_hbm, yb_hbm, src_hbm, dst_hbm, outa_hbm, outb_hbm, idxs, idxd,
           r0, r1, r2, r3, zbuf, acc, *sems):
    c = lax.axis_index("c")
    s = lax.axis_index("s")
    wid = c * NS + s
    _fill(zbuf, ZR, DH2, 0.0)
    pltpu.sync_copy(src_hbm.at[pl.ds(wid * CPW, CPW)], idxs)
    pltpu.sync_copy(dst_hbm.at[pl.ds(wid * CPW, CPW)], idxd)
    bufs = (r0, r1, r2, r3)
    for y_hbm, out_hbm in ((ya_hbm, outa_hbm), (yb_hbm, outb_hbm)):
        _zero_acc(acc, zbuf, s)
        plsc.subcore_barrier()
        _scatter_pass(y_hbm, idxs, idxd, bufs, acc, sems[:4], sems[4:])
        plsc.subcore_barrier()
        pltpu.sync_copy(
            acc.at[pl.ds(s * NPT, NPT)],
            out_hbm.at[pl.ds(c * NPAD + s * NPT, NPT)],
        )
        plsc.subcore_barrier()


@functools.partial(
    pl.kernel,
    out_type=jax.ShapeDtypeStruct((NC * NPAD, D2P), jnp.float32),
    mesh=_MESH,
    scratch_types=[
        pltpu.VMEM((CPW, K), jnp.int32),
        pltpu.VMEM((CPW, K), jnp.int32),
        pltpu.VMEM((K, D2P), jnp.float32),
        pltpu.VMEM((K, D2P), jnp.float32),
        pltpu.VMEM((K, D2P), jnp.float32),
        pltpu.VMEM((K, D2P), jnp.float32),
        pltpu.VMEM((ZR, D2P), jnp.float32),
        pltpu.VMEM_SHARED((NPAD, D2P), jnp.float32),
    ] + [pltpu.SemaphoreType.DMA] * 8,
    compiler_params=_SC_PARAMS,
)
def _prop2(y_hbm, src_hbm, dst_hbm, out_hbm, idxs, idxd, r0, r1, r2, r3, zbuf,
           acc, *sems):
    c = lax.axis_index("c")
    s = lax.axis_index("s")
    wid = c * NS + s
    _fill(zbuf, ZR, D2P, 0.0)
    _zero_acc(acc, zbuf, s)
    plsc.subcore_barrier()
    pltpu.sync_copy(src_hbm.at[pl.ds(wid * CPW, CPW)], idxs)
    pltpu.sync_copy(dst_hbm.at[pl.ds(wid * CPW, CPW)], idxd)
    _scatter_pass(y_hbm, idxs, idxd, (r0, r1, r2, r3), acc, sems[:4], sems[4:])
    plsc.subcore_barrier()
    pltpu.sync_copy(
        acc.at[pl.ds(s * NPT, NPT)], out_hbm.at[pl.ds(c * NPAD + s * NPT, NPT)]
    )


R = 1000


def _tc1_body(dega_ref, degb_ref, x_ref, w1_ref, ya_ref, yb_ref, dinv_ref):
    deg = dega_ref[:, 0:1] + degb_ref[:, 0:1]
    dinv = lax.rsqrt(deg)
    xw = jnp.dot(x_ref[...], w1_ref[...], preferred_element_type=jnp.float32)
    y1 = xw * dinv
    ya_ref[...] = y1[:, :DH2]
    yb_ref[...] = y1[:, DH2:]
    dinv_ref[...] = jnp.broadcast_to(dinv, (R, 8))


def _tc1(dega, degb, x, w1):
    return pl.pallas_call(
        _tc1_body,
        grid=(N // R,),
        in_specs=[
            pl.BlockSpec((R, 16), lambda i: (i, 0)),
            pl.BlockSpec((R, 16), lambda i: (i, 0)),
            pl.BlockSpec((R, D_IN), lambda i: (i, 0)),
            pl.BlockSpec((D_IN, D_H), lambda i: (0, 0)),
        ],
        out_specs=[
            pl.BlockSpec((R, DH2), lambda i: (i, 0)),
            pl.BlockSpec((R, DH2), lambda i: (i, 0)),
            pl.BlockSpec((R, 8), lambda i: (i, 0)),
        ],
        out_shape=[
            jax.ShapeDtypeStruct((N, DH2), jnp.float32),
            jax.ShapeDtypeStruct((N, DH2), jnp.float32),
            jax.ShapeDtypeStruct((N, 8), jnp.float32),
        ],
    )(dega, degb, x, w1)


def _tc2_body(za_a, za_b, zb_a, zb_b, ya_ref, yb_ref, dinv_ref, b1_ref, w2_ref,
              y2_ref):
    dinv = dinv_ref[:, 0:1]
    zlo = za_a[...] + za_b[...] + ya_ref[...]
    zhi = zb_a[...] + zb_b[...] + yb_ref[...]
    z = jnp.concatenate([zlo, zhi], axis=1)
    h = jnp.maximum(z * dinv + b1_ref[...], 0.0)
    y2_ref[...] = jnp.dot(h, w2_ref[...], preferred_element_type=jnp.float32) * dinv


def _tc2(acca, accb, ya, yb, dinv, b1, w2p):
    half = pl.BlockSpec((R, DH2), lambda i: (i, 0))
    return pl.pallas_call(
        _tc2_body,
        grid=(N // R,),
        in_specs=[
            half, half, half, half, half, half,
            pl.BlockSpec((R, 8), lambda i: (i, 0)),
            pl.BlockSpec((1, D_H), lambda i: (0, 0)),
            pl.BlockSpec((D_H, D2P), lambda i: (0, 0)),
        ],
        out_specs=pl.BlockSpec((R, D2P), lambda i: (i, 0)),
        out_shape=jax.ShapeDtypeStruct((N, D2P), jnp.float32),
    )(acca[:N], acca[NPAD:NPAD + N], accb[:N], accb[NPAD:NPAD + N], ya, yb,
      dinv, b1, w2p)


def _tc3_body(acca_ref, accb_ref, y2_ref, dinv_ref, b2_ref, out_ref):
    dinv = dinv_ref[:, 0:1]
    z = (acca_ref[...] + accb_ref[...] + y2_ref[...]) * dinv
    o = z[:, :D_OUT] + b2_ref[...]
    m = jnp.max(o, axis=1, keepdims=True)
    e = jnp.exp(o - m)
    out_ref[...] = e / jnp.sum(e, axis=1, keepdims=True)


def _tc3(acc2, y2, dinv, b2):
    return pl.pallas_call(
        _tc3_body,
        grid=(N // R,),
        in_specs=[
            pl.BlockSpec((R, D2P), lambda i: (i, 0)),
            pl.BlockSpec((R, D2P), lambda i: (i, 0)),
            pl.BlockSpec((R, D2P), lambda i: (i, 0)),
            pl.BlockSpec((R, 8), lambda i: (i, 0)),
            pl.BlockSpec((1, D_OUT), lambda i: (0, 0)),
        ],
        out_specs=pl.BlockSpec((R, D_OUT), lambda i: (i, 0)),
        out_shape=jax.ShapeDtypeStruct((N, D_OUT), jnp.float32),
    )(acc2[:N], acc2[NPAD:NPAD + N], y2, dinv, b2)


def kernel(x, edge_index, W1, b1, W2, b2):
    src = jnp.concatenate([edge_index[0], jnp.asarray(_PAD_SRC)])
    dst = jnp.concatenate([edge_index[1], jnp.asarray(_PAD_DST)])
    src2 = src.reshape(NW * CPW, K)
    dst2 = dst.reshape(NW * CPW, K)
    w2p = jnp.pad(W2, ((0, 0), (0, D2P - D_OUT)))
    b1r = b1.reshape(1, D_H)
    b2r = b2.reshape(1, D_OUT)

    degp = _deg_kernel(dst2)
    ya, yb, dinv = _tc1(degp[:N], degp[NPAD:NPAD + N], x, W1)
    acc1a, acc1b = _prop1(ya, yb, src2, dst2)
    y2 = _tc2(acc1a, acc1b, ya, yb, dinv, b1r, w2p)
    acc2 = _prop2(y2, src2, dst2)
    return _tc3(acc2, y2, dinv, b2r)


# speedup vs baseline: 27.6542x; 1.0443x over previous
import functools

import jax
import jax.numpy as jnp
import numpy as np
from jax import lax
from jax.experimental import pallas as pl
from jax.experimental.pallas import tpu as pltpu
from jax.experimental.pallas import tpu_sc as plsc

N = 10000
E = 320000
D_IN = 128
D_H = 128
DH2 = 64
D_OUT = 40
D2P = 48

NC = 2
NS = 16
NW = NC * NS
K = 128
CPW = 80
EP = NW * CPW * K
NPAD = 10112
NPT = NPAD // NS
ZR = 158

_MESH = plsc.VectorSubcoreMesh(
    core_axis_name="c", subcore_axis_name="s", num_cores=NC, num_subcores=NS
)
_SC_PARAMS = pltpu.CompilerParams(use_tc_tiling_on_sc=False)

_PAD_DST = np.asarray(N + (np.arange(EP - E) % (NPAD - N)), np.int32)
_PAD_SRC = np.asarray((np.arange(EP - E) * 37) % N, np.int32)


def _fill(ref, nrows, ncols, value):
    val = jnp.full((16,), value, jnp.float32)

    def body(i, _):
        for j in range(ncols // 16):
            ref[i, pl.ds(j * 16, 16)] = val
        return 0

    lax.fori_loop(0, nrows, body, 0)


def _zero_acc(acc, zbuf, s):
    for r in range(NPT // ZR):
        pltpu.sync_copy(zbuf, acc.at[pl.ds(s * NPT + r * ZR, ZR)])


NB = 4


def _scatter_pass(y_hbm, idxs, idxd, bufs, acc, sgs, sss):
    def body(t, _):
        gs = [
            pltpu.async_copy(y_hbm.at[idxs.at[NB * t + u]], bufs[u], sgs[u])
            for u in range(NB)
        ]
        ss = []
        for u in range(NB):
            gs[u].wait()
            ss.append(
                pltpu.async_copy(bufs[u], acc.at[idxd.at[NB * t + u]], sss[u],
                                 add=True)
            )
        for s_ in ss:
            s_.wait()
        return 0

    lax.fori_loop(0, CPW // NB, body, 0)


@functools.partial(
    pl.kernel,
    out_type=jax.ShapeDtypeStruct((NC * NPAD, 16), jnp.float32),
    mesh=_MESH,
    scratch_types=[
        pltpu.VMEM((CPW, K), jnp.int32),
        pltpu.VMEM((K, 16), jnp.float32),
        pltpu.VMEM((ZR, 16), jnp.float32),
        pltpu.VMEM_SHARED((NPAD, 16), jnp.float32),
        pltpu.SemaphoreType.DMA,
    ],
    compiler_params=_SC_PARAMS,
)
def _deg_kernel(dst_hbm, out_hbm, idxd, ones_v, zbuf, acc, sem):
    c = lax.axis_index("c")
    s = lax.axis_index("s")
    wid = c * NS + s
    _fill(ones_v, K, 16, 1.0)
    _fill(zbuf, ZR, 16, 0.0)
    _zero_acc(acc, zbuf, s)
    plsc.subcore_barrier()
    pltpu.sync_copy(dst_hbm.at[pl.ds(wid * CPW, CPW)], idxd)

    def body(t, _):
        ds_ = [
            pltpu.async_copy(ones_v, acc.at[idxd.at[4 * t + u]], sem, add=True)
            for u in range(4)
        ]
        for d_ in ds_:
            d_.wait()
        return 0

    lax.fori_loop(0, CPW // 4, body, 0)
    plsc.subcore_barrier()
    pltpu.sync_copy(
        acc.at[pl.ds(s * NPT, NPT)], out_hbm.at[pl.ds(c * NPAD + s * NPT, NPT)]
    )


@functools.partial(
    pl.kernel,
    out_type=[
        jax.ShapeDtypeStruct((NC * NPAD, DH2), jnp.float32),
        jax.ShapeDtypeStruct((NC * NPAD, DH2), jnp.float32),
    ],
    mesh=_MESH,
    scratch_types=[
        pltpu.VMEM((CPW, K), jnp.int32),
        pltpu.VMEM((CPW, K), jnp.int32),
        pltpu.VMEM((K, DH2), jnp.float32),
        pltpu.VMEM((K, DH2), jnp.float32),
        pltpu.VMEM((K, DH2), jnp.float32),
        pltpu.VMEM((K, DH2), jnp.float32),
        pltpu.VMEM((ZR, DH2), jnp.float32),
        pltpu.VMEM_SHARED((NPAD, DH2), jnp.float32),
    ] + [pltpu.SemaphoreType.DMA] * 8,
    compiler_params=_SC_PARAMS,
)
def _prop1(ya_hbm, yb_hbm, src_hbm, dst_hbm, outa_hbm, outb_hbm, idxs, idxd,
           r0, r1, r2, r3, zbuf, acc, *sems):
    c = lax.axis_index("c")
    s = lax.axis_index("s")
    wid = c * NS + s
    _fill(zbuf, ZR, DH2, 0.0)
    pltpu.sync_copy(src_hbm.at[pl.ds(wid * CPW, CPW)], idxs)
    pltpu.sync_copy(dst_hbm.at[pl.ds(wid * CPW, CPW)], idxd)
    bufs = (r0, r1, r2, r3)
    for y_hbm, out_hbm in ((ya_hbm, outa_hbm), (yb_hbm, outb_hbm)):
        _zero_acc(acc, zbuf, s)
        plsc.subcore_barrier()
        _scatter_pass(y_hbm, idxs, idxd, bufs, acc, sems[:4], sems[4:])
        plsc.subcore_barrier()
        pltpu.sync_copy(
            acc.at[pl.ds(s * NPT, NPT)],
            out_hbm.at[pl.ds(c * NPAD + s * NPT, NPT)],
        )
        plsc.subcore_barrier()


@functools.partial(
    pl.kernel,
    out_type=jax.ShapeDtypeStruct((NC * NPAD, D2P), jnp.float32),
    mesh=_MESH,
    scratch_types=[
        pltpu.VMEM((CPW, K), jnp.int32),
        pltpu.VMEM((CPW, K), jnp.int32),
        pltpu.VMEM((K, D2P), jnp.float32),
        pltpu.VMEM((K, D2P), jnp.float32),
        pltpu.VMEM((K, D2P), jnp.float32),
        pltpu.VMEM((K, D2P), jnp.float32),
        pltpu.VMEM((ZR, D2P), jnp.float32),
        pltpu.VMEM_SHARED((NPAD, D2P), jnp.float32),
    ] + [pltpu.SemaphoreType.DMA] * 8,
    compiler_params=_SC_PARAMS,
)
def _prop2(y_hbm, src_hbm, dst_hbm, out_hbm, idxs, idxd, r0, r1, r2, r3, zbuf,
           acc, *sems):
    c = lax.axis_index("c")
    s = lax.axis_index("s")
    wid = c * NS + s
    _fill(zbuf, ZR, D2P, 0.0)
    _zero_acc(acc, zbuf, s)
    plsc.subcore_barrier()
    pltpu.sync_copy(src_hbm.at[pl.ds(wid * CPW, CPW)], idxs)
    pltpu.sync_copy(dst_hbm.at[pl.ds(wid * CPW, CPW)], idxd)
    _scatter_pass(y_hbm, idxs, idxd, (r0, r1, r2, r3), acc, sems[:4], sems[4:])
    plsc.subcore_barrier()
    pltpu.sync_copy(
        acc.at[pl.ds(s * NPT, NPT)], out_hbm.at[pl.ds(c * NPAD + s * NPT, NPT)]
    )


R = 632
GRID = NPAD // R


def _tc1_body(dega_ref, degb_ref, x_ref, w1_ref, ya_ref, yb_ref, dinv_ref):
    deg = dega_ref[:, 0:1] + degb_ref[:, 0:1] + 1.0
    dinv = lax.rsqrt(deg)
    xw = jnp.dot(x_ref[...], w1_ref[...], preferred_element_type=jnp.float32)
    y1 = xw * dinv
    ya_ref[...] = y1[:, :DH2]
    yb_ref[...] = y1[:, DH2:]
    dinv_ref[...] = jnp.broadcast_to(dinv, (R, 8))


def _tc1(degp, xpad, w1):
    return pl.pallas_call(
        _tc1_body,
        grid=(GRID,),
        in_specs=[
            pl.BlockSpec((R, 16), lambda i: (i, 0)),
            pl.BlockSpec((R, 16), lambda i: (i + GRID, 0)),
            pl.BlockSpec((R, D_IN), lambda i: (i, 0)),
            pl.BlockSpec((D_IN, D_H), lambda i: (0, 0)),
        ],
        out_specs=[
            pl.BlockSpec((R, DH2), lambda i: (i, 0)),
            pl.BlockSpec((R, DH2), lambda i: (i, 0)),
            pl.BlockSpec((R, 8), lambda i: (i, 0)),
        ],
        out_shape=[
            jax.ShapeDtypeStruct((NPAD, DH2), jnp.float32),
            jax.ShapeDtypeStruct((NPAD, DH2), jnp.float32),
            jax.ShapeDtypeStruct((NPAD, 8), jnp.float32),
        ],
    )(degp, degp, xpad, w1)


def _tc2_body(za_a, za_b, zb_a, zb_b, ya_ref, yb_ref, dinv_ref, b1_ref, w2_ref,
              y2_ref):
    dinv = dinv_ref[:, 0:1]
    zlo = za_a[...] + za_b[...] + ya_ref[...]
    zhi = zb_a[...] + zb_b[...] + yb_ref[...]
    z = jnp.concatenate([zlo, zhi], axis=1)
    h = jnp.maximum(z * dinv + b1_ref[...], 0.0)
    y2_ref[...] = jnp.dot(h, w2_ref[...], preferred_element_type=jnp.float32) * dinv


def _tc2(acca, accb, ya, yb, dinv, b1, w2p):
    halfa = pl.BlockSpec((R, DH2), lambda i: (i, 0))
    halfb = pl.BlockSpec((R, DH2), lambda i: (i + GRID, 0))
    return pl.pallas_call(
        _tc2_body,
        grid=(GRID,),
        in_specs=[
            halfa, halfb, halfa, halfb, halfa, halfa,
            pl.BlockSpec((R, 8), lambda i: (i, 0)),
            pl.BlockSpec((1, D_H), lambda i: (0, 0)),
            pl.BlockSpec((D_H, D2P), lambda i: (0, 0)),
        ],
        out_specs=pl.BlockSpec((R, D2P), lambda i: (i, 0)),
        out_shape=jax.ShapeDtypeStruct((NPAD, D2P), jnp.float32),
    )(acca, acca, accb, accb, ya, yb, dinv, b1, w2p)


def _tc3_body(acca_ref, accb_ref, y2_ref, dinv_ref, b2_ref, out_ref):
    dinv = dinv_ref[:, 0:1]
    z = (acca_ref[...] + accb_ref[...] + y2_ref[...]) * dinv
    o = z[:, :D_OUT] + b2_ref[...]
    m = jnp.max(o, axis=1, keepdims=True)
    e = jnp.exp(o - m)
    out_ref[...] = e / jnp.sum(e, axis=1, keepdims=True)


def _tc3(acc2, y2, dinv, b2):
    return pl.pallas_call(
        _tc3_body,
        grid=(GRID,),
        in_specs=[
            pl.BlockSpec((R, D2P), lambda i: (i, 0)),
            pl.BlockSpec((R, D2P), lambda i: (i + GRID, 0)),
            pl.BlockSpec((R, D2P), lambda i: (i, 0)),
            pl.BlockSpec((R, 8), lambda i: (i, 0)),
            pl.BlockSpec((1, D_OUT), lambda i: (0, 0)),
        ],
        out_specs=pl.BlockSpec((R, D_OUT), lambda i: (i, 0)),
        out_shape=jax.ShapeDtypeStruct((NPAD, D_OUT), jnp.float32),
    )(acc2, acc2, y2, dinv, b2)


def kernel(x, edge_index, W1, b1, W2, b2):
    src = jnp.concatenate([edge_index[0], jnp.asarray(_PAD_SRC)])
    dst = jnp.concatenate([edge_index[1], jnp.asarray(_PAD_DST)])
    src2 = src.reshape(NW * CPW, K)
    dst2 = dst.reshape(NW * CPW, K)
    w2p = jnp.pad(W2, ((0, 0), (0, D2P - D_OUT)))
    b1r = b1.reshape(1, D_H)
    b2r = b2.reshape(1, D_OUT)

    xpad = jnp.pad(x, ((0, NPAD - N), (0, 0)))
    degp = _deg_kernel(dst2)
    ya, yb, dinv = _tc1(degp, xpad, W1)
    acc1a, acc1b = _prop1(ya, yb, src2, dst2)
    y2 = _tc2(acc1a, acc1b, ya, yb, dinv, b1r, w2p)
    acc2 = _prop2(y2, src2, dst2)
    return _tc3(acc2, y2, dinv, b2r)[:N]
